# Initial kernel scaffold; baseline (speedup 1.0000x reference)
import jax
import jax.numpy as jnp
from jax.experimental import pallas as pl


def kernel(x_in, start_w, skip0_w, gtu_wp, gtu_wq, theta, cheb, start_b, skip0_b, gtu_bp, gtu_bq, l0_skip_w, l0_skip_b, l0_M1_0, l0_M1_1, l0_M1_2, l0_M2_0, l0_M2_1, l0_M2_2, l0_bF1, l0_bF2, l0_U1k, l0_U2, l0_U3k, l0_Ve, l0_be, l0_W1row, l0_W2, l0_W3k, l0_Vs, l0_bs, l0_R8T, l0_Krep, l0_RNl, l0_MaskTT, l1_skip_w, l1_skip_b, l1_M1_0, l1_M1_1, l1_M1_2, l1_M2_0, l1_M2_1, l1_M2_2, l1_bF1, l1_bF2, l1_U1k, l1_U2, l1_U3k, l1_Ve, l1_be, l1_W1row, l1_W2, l1_W3k, l1_Vs, l1_bs, l1_R8T, l1_Krep, l1_RNl, l1_MaskTT, l2_skip_w, l2_skip_b, l2_M1_0, l2_M1_1, l2_M1_2, l2_M2_0, l2_M2_1, l2_M2_2, l2_bF1, l2_bF2, l2_U1k, l2_U2, l2_U3k, l2_Ve, l2_be, l2_W1row, l2_W2, l2_W3k, l2_Vs, l2_bs, l2_R8T, l2_Krep, l2_RNl, l2_MaskTT, skipE_w, end1_w, end2_w, skipE_b, end1_b, end2_b):
    raise NotImplementedError("write your pallas kernel here")


# speedup vs baseline: 1.0090x; 1.0090x over previous
import functools

import jax
import jax.numpy as jnp
from jax.experimental import pallas as pl
from jax.experimental.pallas import tpu as pltpu

F32 = jnp.float32
GTU_KS = (3, 5, 7)

B = 8
N = 8
BN = B * N
CIN = 2
C = 32
SC = 64
EC = 128
SEQ = 12
T0 = 19
NLAYERS = 3
K = 3
OUT_DIM = 12
EPS = 1e-5


def _dot(a, b):
    return jnp.dot(a, b, preferred_element_type=F32)


def _dot_bt(a, b):
    return jax.lax.dot_general(a, b, (((1,), (1,)), ((), ())),
                               preferred_element_type=F32)


def _softmax0(x):
    m = jnp.max(x, axis=0, keepdims=True)
    e = jnp.exp(x - m)
    return e / jnp.sum(e, axis=0, keepdims=True)


def _gtu_bank(X, T, wpq, bp, bq):
    outs = []
    tap = 0
    for j, k in enumerate(GTU_KS):
        rows = (T - k + 1) * N
        acc = jnp.zeros((rows, 2 * C), F32)
        for dt in range(k):
            acc = acc + _dot(X[dt * N: dt * N + rows, :],
                             wpq[(tap + dt) * C:(tap + dt + 1) * C, :])
        tap += k
        p = acc[:, :C] + bp[:, j * C:(j + 1) * C]
        q = acc[:, C:] + bq[:, j * C:(j + 1) * C]
        outs.append(jnp.tanh(p) * jax.nn.sigmoid(q))
    return outs


def _body(idx, nin, *refs):
    o_ref = refs[nin]

    def R(name):
        return refs[idx[name]]

    wpq = R("wpq")[...]
    bp = R("gtu_bp")[...]
    bq = R("gtu_bq")[...]
    cheb = R("cheb")[...]
    theta = R("theta")[...]

    x = _dot(R("xc")[0], R("start_w")[...]) + R("start_b")[...]
    skip_last = _dot(R("xr")[0], R("skip0_w")[...]) + R("skip0_b")[...]

    T = T0
    for i in range(NLAYERS):
        T_out = T - 6
        kw = T_out
        to_s = T - kw + 1
        residual = x

        g = _gtu_bank(x, T, wpq, bp, bq)
        tc = R("l%d_bF1" % i)[0]
        for j in range(3):
            tc = tc + _dot(R("l%d_K1_%d" % (i, j))[...], g[j])
        x_new = jnp.maximum(x + tc, 0.0)

        s = jnp.zeros((N, SC), F32) + R("l%d_skip_b" % i)[...]
        skw = R("l%d_skip_w" % i)[...]
        for dt in range(kw):
            r0 = (to_s - 1 + dt) * N
            s = s + _dot(x_new[r0:r0 + N, :], skw[dt * C:(dt + 1) * C, :])
        skip_last = skip_last + s

        r8t = R("l%d_R8T" % i)[...]
        rnl = R("l%d_RNl" % i)[...]
        lhs1 = _dot(_dot(R("l%d_Ku1" % i)[...], x_new), R("l%d_U2" % i)[...])
        v3 = _dot_bt(R("l%d_u3row" % i)[...], x_new)
        rhs1t = _dot(r8t * v3, rnl)
        prod1 = _dot_bt(lhs1, rhs1t)
        e = _dot(R("l%d_Ve" % i)[...],
                 jax.nn.sigmoid(prod1 + R("l%d_be" % i)[...]))
        t_att = _softmax0(e)

        w1t = _dot(R("l%d_W1row" % i)[...], t_att)
        k1 = R("l%d_Krep" % i)[...] * _dot(w1t, r8t)
        r = _dot(k1, x_new)
        lhs2 = _dot(r, R("l%d_W2" % i)[...])
        vw3 = _dot_bt(R("l%d_w3row" % i)[...], x_new)
        rhs2 = _dot(t_att, _dot(r8t * vw3, rnl))
        prod2 = _dot(lhs2, rhs2)
        s_f = _dot(R("l%d_Vs" % i)[...],
                   jax.nn.sigmoid(prod2 + R("l%d_bs" % i)[...]))
        s_att = _softmax0(s_f)

        mask = R("l%d_MaskTT" % i)[...]
        acc = jnp.zeros((T * N, C), F32)
        for kk in range(K):
            a = cheb[kk * N:(kk + 1) * N, :] * s_att
            big = _dot_bt(_dot_bt(rnl, a), rnl) * mask
            acc = acc + _dot(_dot(big, x_new), theta[kk * C:(kk + 1) * C, :])
        xg = jnp.maximum(acc, 0.0)

        g = _gtu_bank(xg, T, wpq, bp, bq)
        tc2 = R("l%d_bF2" % i)[0]
        for j in range(3):
            tc2 = tc2 + _dot(R("l%d_K2_%d" % (i, j))[...], g[j])
        off = (T - T_out) * N
        xn2 = jnp.maximum(xg[off:, :] + tc2, 0.0) + residual[off:, :]
        cnt = float(T_out * N * C)
        mu = jnp.sum(xn2) / cnt
        d = xn2 - mu
        var = jnp.sum(d * d) / cnt
        x = d * jax.lax.rsqrt(var + EPS)
        T = T_out

    sk = _dot(x, R("skipE_w")[...]) + R("skipE_b")[...] + skip_last
    h = jnp.maximum(sk, 0.0)
    h = jnp.maximum(_dot(h, R("end1_w")[...]) + R("end1_b")[...], 0.0)
    o_ref[0] = _dot(h, R("end2_w")[...]) + R("end2_b")[...]


def kernel(x_in, start_w, skip0_w, gtu_wp, gtu_wq, theta, cheb, start_b, skip0_b, gtu_bp, gtu_bq, l0_skip_w, l0_skip_b, l0_M1_0, l0_M1_1, l0_M1_2, l0_M2_0, l0_M2_1, l0_M2_2, l0_bF1, l0_bF2, l0_U1k, l0_U2, l0_U3k, l0_Ve, l0_be, l0_W1row, l0_W2, l0_W3k, l0_Vs, l0_bs, l0_R8T, l0_Krep, l0_RNl, l0_MaskTT, l1_skip_w, l1_skip_b, l1_M1_0, l1_M1_1, l1_M1_2, l1_M2_0, l1_M2_1, l1_M2_2, l1_bF1, l1_bF2, l1_U1k, l1_U2, l1_U3k, l1_Ve, l1_be, l1_W1row, l1_W2, l1_W3k, l1_Vs, l1_bs, l1_R8T, l1_Krep, l1_RNl, l1_MaskTT, l2_skip_w, l2_skip_b, l2_M1_0, l2_M1_1, l2_M1_2, l2_M2_0, l2_M2_1, l2_M2_2, l2_bF1, l2_bF2, l2_U1k, l2_U2, l2_U3k, l2_Ve, l2_be, l2_W1row, l2_W2, l2_W3k, l2_Vs, l2_bs, l2_R8T, l2_Krep, l2_RNl, l2_MaskTT, skipE_w, end1_w, end2_w, skipE_b, end1_b, end2_b):
    lraw = [
        dict(skip_w=l0_skip_w, skip_b=l0_skip_b,
             M1=(l0_M1_0, l0_M1_1, l0_M1_2), M2=(l0_M2_0, l0_M2_1, l0_M2_2),
             bF1=l0_bF1, bF2=l0_bF2, U1k=l0_U1k, U2=l0_U2, U3k=l0_U3k,
             Ve=l0_Ve, be=l0_be, W1row=l0_W1row, W2=l0_W2, W3k=l0_W3k,
             Vs=l0_Vs, bs=l0_bs, R8T=l0_R8T, Krep=l0_Krep, RNl=l0_RNl,
             MaskTT=l0_MaskTT),
        dict(skip_w=l1_skip_w, skip_b=l1_skip_b,
             M1=(l1_M1_0, l1_M1_1, l1_M1_2), M2=(l1_M2_0, l1_M2_1, l1_M2_2),
             bF1=l1_bF1, bF2=l1_bF2, U1k=l1_U1k, U2=l1_U2, U3k=l1_U3k,
             Ve=l1_Ve, be=l1_be, W1row=l1_W1row, W2=l1_W2, W3k=l1_W3k,
             Vs=l1_Vs, bs=l1_bs, R8T=l1_R8T, Krep=l1_Krep, RNl=l1_RNl,
             MaskTT=l1_MaskTT),
        dict(skip_w=l2_skip_w, skip_b=l2_skip_b,
             M1=(l2_M1_0, l2_M1_1, l2_M1_2), M2=(l2_M2_0, l2_M2_1, l2_M2_2),
             bF1=l2_bF1, bF2=l2_bF2, U1k=l2_U1k, U2=l2_U2, U3k=l2_U3k,
             Ve=l2_Ve, be=l2_be, W1row=l2_W1row, W2=l2_W2, W3k=l2_W3k,
             Vs=l2_Vs, bs=l2_bs, R8T=l2_R8T, Krep=l2_Krep, RNl=l2_RNl,
             MaskTT=l2_MaskTT),
    ]

    xp = jnp.pad(x_in, ((0, 0), (0, 0), (0, 0), (T0 - SEQ, 0)))
    xc = xp.transpose(0, 3, 2, 1).reshape(B, T0 * N, CIN)
    xr = xp.transpose(0, 2, 3, 1).reshape(B, N, T0 * CIN)
    wpq = jnp.concatenate([gtu_wp, gtu_wq], axis=1)
    eyeN = jnp.eye(N, dtype=F32)

    args = []
    specs = []
    idx = {}

    def add(name, arr, per_sample=False):
        idx[name] = len(args)
        args.append(arr)
        if per_sample:
            blk = (1,) + arr.shape[1:]
            specs.append(pl.BlockSpec(blk, lambda i: (i,) + (0,) * (arr.ndim - 1)))
        else:
            nd = arr.ndim
            specs.append(pl.BlockSpec(arr.shape, lambda i, _n=nd: (0,) * _n))

    add("xc", xc, per_sample=True)
    add("xr", xr, per_sample=True)
    for nm, arr in (("start_w", start_w), ("start_b", start_b),
                    ("skip0_w", skip0_w), ("skip0_b", skip0_b),
                    ("wpq", wpq), ("gtu_bp", gtu_bp), ("gtu_bq", gtu_bq),
                    ("cheb", cheb), ("theta", theta),
                    ("skipE_w", skipE_w), ("skipE_b", skipE_b),
                    ("end1_w", end1_w), ("end1_b", end1_b),
                    ("end2_w", end2_w), ("end2_b", end2_b)):
        add(nm, arr)

    T = T0
    for i, lp in enumerate(lraw):
        T_out = T - 6
        for j in range(3):
            a1 = lp["M1"][j][::BN, ::BN]
            a2 = lp["M2"][j][::BN, ::BN]
            add("l%d_K1_%d" % (i, j), jnp.kron(a1, eyeN))
            add("l%d_K2_%d" % (i, j), jnp.kron(a2, eyeN))
        add("l%d_bF1" % i,
            lp["bF1"].reshape(T, B, N).transpose(1, 0, 2).reshape(B, T * N, 1),
            per_sample=True)
        add("l%d_bF2" % i,
            lp["bF2"].reshape(T_out, B, N).transpose(1, 0, 2)
            .reshape(B, T_out * N, 1), per_sample=True)
        u1 = lp["U1k"][::C, 0]
        add("l%d_Ku1" % i, jnp.kron(jnp.eye(T, dtype=F32), u1[None, :]))
        add("l%d_u3row" % i, lp["U3k"][0:C, 0][None, :])
        add("l%d_w3row" % i, lp["W3k"][0:C, 0][None, :])
        for nm in ("skip_w", "skip_b", "U2", "Ve", "be", "W1row", "W2",
                   "Vs", "bs", "R8T", "Krep", "RNl", "MaskTT"):
            add("l%d_%s" % (i, nm), lp[nm])
        T = T_out

    nin = len(args)
    out = pl.pallas_call(
        functools.partial(_body, idx, nin),
        out_shape=jax.ShapeDtypeStruct((B, N, OUT_DIM), F32),
        grid=(B,),
        in_specs=specs,
        out_specs=pl.BlockSpec((1, N, OUT_DIM), lambda i: (i, 0, 0)),
        compiler_params=pltpu.CompilerParams(dimension_semantics=("parallel",)),
    )(*args)
    return out.transpose(0, 2, 1)[..., None]
